# baseline (speedup 1.0000x reference)
import functools

import jax
import jax.numpy as jnp
from jax import lax
from jax.experimental import pallas as pl
from jax.experimental.pallas import tpu as pltpu
from jax.experimental.pallas import tpu_sc as plsc

B, L, D = 4096, 200, 16
P = B * L
G = 8
ROWS = P // G
NC, NS = 2, 16
NW = NC * NS
QV = P // NW
CH = 1024
NCHUNK = QV // CH
BB = 128
RB = BB * L // G
GRID = B // BB


N_TAB = 1000000
RC = 32768
RC8 = RC // 8
RSH = RC8.bit_length() - 1
RGRID = -(-N_TAB // RC)
SLOTS = RGRID * RC


def _repack_body(tab_t, x_ref):
  blk = tab_t[...]
  cat = jnp.concatenate(
      [blk[:, g * RC8:(g + 1) * RC8] for g in range(8)], axis=0)
  x_ref[...] = cat.T


def _repack1(tab_t):
  return pl.pallas_call(
      _repack_body,
      grid=(RGRID,),
      in_specs=[pl.BlockSpec((D, RC), lambda i: (0, i))],
      out_specs=pl.BlockSpec((RC8, 8 * D), lambda i: (i, 0)),
      out_shape=jax.ShapeDtypeStruct((SLOTS // 8, 8 * D), jnp.float32),
      compiler_params=pltpu.CompilerParams(
          dimension_semantics=("arbitrary",)),
  )(tab_t)


def _sigma(vec):
  return ((vec & (-RC)) | ((vec & (RC8 - 1)) << 3)
          | ((vec & (RC - 1)) >> RSH))


def _sc_v_body(huv, xv, out_euv, idx_v, idx2_v, rows_v, sem):
  wid = lax.axis_index("s") * NC + lax.axis_index("c")

  @pl.loop(0, NCHUNK)
  def _(i):
    row0 = wid * (QV // 128) + i * (CH // 128)
    base = wid * QV + i * CH
    pltpu.sync_copy(huv.at[pl.ds(row0, CH // 128)], idx_v)
    for j in range(CH // 128):
      for k in range(8):
        idx2_v[j, pl.ds(k * 16, 16)] = _sigma(idx_v[j, pl.ds(k * 16, 16)])
    copies = [
        pltpu.async_copy(xv.at[idx2_v.at[j]],
                         rows_v.at[pl.ds(j * 128, 128)], sem)
        for j in range(CH // 128)
    ]
    for c in copies:
      c.wait()
    pltpu.sync_copy(rows_v, out_euv.at[pl.ds(base, CH)])


def _sc_u_body(nodes, xu, out_uv, nidx_v, nidx2_v, urows_v, sem):
  wid = lax.axis_index("s") * NC + lax.axis_index("c")
  pltpu.sync_copy(nodes.at[pl.ds(wid * (B // NW), B // NW)], nidx_v)
  for k in range(8):
    nidx2_v[pl.ds(k * 16, 16)] = _sigma(nidx_v[pl.ds(k * 16, 16)])
  pltpu.async_copy(xu.at[nidx2_v], urows_v, sem).wait()
  pltpu.sync_copy(urows_v, out_uv.at[pl.ds(wid * (B // NW), B // NW)])


def _sc_mesh():
  return plsc.VectorSubcoreMesh(core_axis_name="c", subcore_axis_name="s",
                                num_cores=NC, num_subcores=NS)


def _sc_gather_v(huv2d, xv):
  return pl.kernel(
      _sc_v_body,
      out_type=jax.ShapeDtypeStruct((P, D), jnp.float32),
      mesh=_sc_mesh(),
      scratch_types=[
          pltpu.VMEM((CH // 128, 128), jnp.int32),
          pltpu.VMEM((CH // 128, 128), jnp.int32),
          pltpu.VMEM((CH, D), jnp.float32),
          pltpu.SemaphoreType.DMA,
      ],
      compiler_params=pltpu.CompilerParams(use_tc_tiling_on_sc=False),
  )(huv2d, xv)


def _sc_gather_u(nodes1d, xu):
  return pl.kernel(
      _sc_u_body,
      out_type=jax.ShapeDtypeStruct((B, D), jnp.float32),
      mesh=_sc_mesh(),
      scratch_types=[
          pltpu.VMEM((B // NW,), jnp.int32),
          pltpu.VMEM((B // NW,), jnp.int32),
          pltpu.VMEM((B // NW, D), jnp.float32),
          pltpu.SemaphoreType.DMA,
      ],
      compiler_params=pltpu.CompilerParams(use_tc_tiling_on_sc=False),
  )(nodes1d, xu)


def _tc_body(euv, hist, uv, bw1a, br2e, bw1b, bw2, a1b, ba1a, ba2, ba3,
             e5, e16, e16d, t16, seg, sege, b1t, b2t, ab1t, ab2t, ab3t,
             out_ref):
  f32 = jnp.float32
  dot = functools.partial(jnp.dot, preferred_element_type=f32)

  ohv = dot(hist[...], e5[...])
  lane = lax.broadcasted_iota(jnp.int32, (RB, 5 * G), 1) % 5
  oh = jnp.where(ohv == lane.astype(f32), 1.0, 0.0)

  brw = dot(br2e[...], bw1b[...])

  x = dot(euv[...], bw1a[...]) + dot(oh, brw) + b1t[...]
  x = jnp.maximum(x, 0.0)
  o = jnp.maximum(dot(x, bw2[...]) + b2t[...], 0.0)

  uvc = dot(uv[...], a1b[...])
  uvrows = dot(dot(sege[...], uvc), e16d[...])
  a1 = jnp.maximum(dot(o, ba1a[...]) + uvrows + ab1t[...], 0.0)
  a2 = jnp.maximum(dot(a1, ba2[...]) + ab2t[...], 0.0)
  lg = dot(a2, ba3[...]) + ab3t[...]
  ex = jnp.exp(lg)

  owg = dot(o * dot(ex, e16[...]), t16[...])
  num = dot(seg[...], owg)
  den = jnp.sum(dot(seg[...], ex), axis=1, keepdims=True)
  out_ref[...] = num / den


def _tc_call(euv_p, hist_p, uv_rep, consts):
  cspecs = [pl.BlockSpec(c.shape, lambda i, nd=c.ndim: (0,) * nd)
            for c in consts]
  return pl.pallas_call(
      _tc_body,
      grid=(GRID,),
      in_specs=[
          pl.BlockSpec((RB, G * D), lambda i: (i, 0)),
          pl.BlockSpec((RB, G), lambda i: (i, 0)),
          pl.BlockSpec((BB, D), lambda i: (i, 0)),
          *cspecs,
      ],
      out_specs=pl.BlockSpec((BB, D), lambda i: (i, 0)),
      out_shape=jax.ShapeDtypeStruct((B, D), jnp.float32),
      compiler_params=pltpu.CompilerParams(
          dimension_semantics=("arbitrary",)),
  )(euv_p, hist_p, uv_rep, *consts)


def _build_consts(W_r1, b_r1, W_r2, b_r2, r2e,
                  att_w1, att_b1, att_w2, att_b2, att_w3, att_b3):
  i8 = jnp.eye(G, dtype=jnp.float32)
  i16 = jnp.eye(D, dtype=jnp.float32)
  kron = jnp.kron
  return [
      kron(i8, W_r1[:D]),
      kron(i8, r2e),
      kron(i8, W_r1[D:]),
      kron(i8, W_r2),
      att_w1[D:],
      kron(i8, att_w1[:D]),
      kron(i8, att_w2),
      kron(i8, att_w3),
      kron(i8, jnp.ones((1, 5), jnp.float32)),
      kron(i8, jnp.ones((1, D), jnp.float32)),
      kron(jnp.ones((1, G), jnp.float32), i16),
      kron(jnp.ones((G, 1), jnp.float32), i16),
      kron(jnp.eye(BB, dtype=jnp.float32),
           jnp.ones((1, L // G), jnp.float32)),
      kron(jnp.eye(BB, dtype=jnp.float32),
           jnp.ones((L // G, 1), jnp.float32)),
      jnp.tile(b_r1, G)[None],
      jnp.tile(b_r2, G)[None],
      jnp.tile(att_b1, G)[None],
      jnp.tile(att_b2, G)[None],
      jnp.tile(att_b3, G)[None],
  ]


def kernel(nodes, history_uv, history_r, v2e, r2e, u2e, W_r1, b_r1, W_r2,
           b_r2, att_w1, att_b1, att_w2, att_b2, att_w3, att_b3):
  huv2d = history_uv.reshape(P // 128, 128).astype(jnp.int32)
  xv = _repack1(v2e.T).reshape(SLOTS, D)
  euv_flat = _sc_gather_v(huv2d, xv)
  xu = _repack1(u2e.T).reshape(SLOTS, D)
  uv_rep = _sc_gather_u(nodes.astype(jnp.int32), xu)
  euv_p = euv_flat.reshape(ROWS, G * D)
  hist_p = history_r.reshape(ROWS, G).astype(jnp.float32)
  consts = _build_consts(W_r1, b_r1, W_r2, b_r2, r2e,
                         att_w1, att_b1, att_w2, att_b2, att_w3, att_b3)
  return _tc_call(euv_p, hist_p, uv_rep, consts)


# speedup vs baseline: 1.0435x; 1.0435x over previous
import functools

import jax
import jax.numpy as jnp
from jax import lax
from jax.experimental import pallas as pl
from jax.experimental.pallas import tpu as pltpu
from jax.experimental.pallas import tpu_sc as plsc

B, L, D = 4096, 200, 16
P = B * L
G = 8
ROWS = P // G
NC, NS = 2, 16
NW = NC * NS
QV = P // NW
CH = 1024
NCHUNK = QV // CH
BB = 128
RB = BB * L // G
GRID = B // BB


N_TAB = 1000000
RC = 32768
RC8 = RC // 8
RSH = RC8.bit_length() - 1
RGRID = -(-N_TAB // RC)
SLOTS = RGRID * RC


def _repack_body(v2e_t, u2e_t, xv_ref, xu_ref):
  for src, dst in ((v2e_t, xv_ref), (u2e_t, xu_ref)):
    blk = src[...]
    cat = jnp.concatenate(
        [blk[:, g * RC8:(g + 1) * RC8] for g in range(8)], axis=0)
    dst[...] = cat.T


def _repack(v2e_t, u2e_t):
  return pl.pallas_call(
      _repack_body,
      grid=(RGRID,),
      in_specs=[
          pl.BlockSpec((D, RC), lambda i: (0, i)),
          pl.BlockSpec((D, RC), lambda i: (0, i)),
      ],
      out_specs=[
          pl.BlockSpec((RC8, 8 * D), lambda i: (i, 0)),
          pl.BlockSpec((RC8, 8 * D), lambda i: (i, 0)),
      ],
      out_shape=[
          jax.ShapeDtypeStruct((SLOTS // 8, 8 * D), jnp.float32),
          jax.ShapeDtypeStruct((SLOTS // 8, 8 * D), jnp.float32),
      ],
      compiler_params=pltpu.CompilerParams(
          dimension_semantics=("arbitrary",)),
  )(v2e_t, u2e_t)


def _sigma(vec):
  return ((vec & (-RC)) | ((vec & (RC8 - 1)) << 3)
          | ((vec & (RC - 1)) >> RSH))


def _sc_body(huv, nodes, xv, xu, out_euv, out_uv,
             idx_v, idx2_v, rows_v, nidx_v, nidx2_v, urows_v, sem):
  wid = lax.axis_index("s") * NC + lax.axis_index("c")

  pltpu.sync_copy(nodes.at[pl.ds(wid * (B // NW), B // NW)], nidx_v)
  for k in range(8):
    nidx2_v[pl.ds(k * 16, 16)] = _sigma(nidx_v[pl.ds(k * 16, 16)])
  pltpu.async_copy(xu.at[nidx2_v], urows_v, sem).wait()
  pltpu.sync_copy(urows_v, out_uv.at[pl.ds(wid * (B // NW), B // NW)])

  @pl.loop(0, NCHUNK)
  def _(i):
    row0 = wid * (QV // 128) + i * (CH // 128)
    base = wid * QV + i * CH
    pltpu.sync_copy(huv.at[pl.ds(row0, CH // 128)], idx_v)
    for j in range(CH // 128):
      for k in range(8):
        idx2_v[j, pl.ds(k * 16, 16)] = _sigma(idx_v[j, pl.ds(k * 16, 16)])
    copies = [
        pltpu.async_copy(xv.at[idx2_v.at[j]],
                         rows_v.at[pl.ds(j * 128, 128)], sem)
        for j in range(CH // 128)
    ]
    for c in copies:
      c.wait()
    pltpu.sync_copy(rows_v, out_euv.at[pl.ds(base, CH)])


def _sc_gather(huv2d, nodes1d, xv, xu):
  mesh = plsc.VectorSubcoreMesh(core_axis_name="c", subcore_axis_name="s",
                                num_cores=NC, num_subcores=NS)
  return pl.kernel(
      _sc_body,
      out_type=[
          jax.ShapeDtypeStruct((P, D), jnp.float32),
          jax.ShapeDtypeStruct((B, D), jnp.float32),
      ],
      mesh=mesh,
      scratch_types=[
          pltpu.VMEM((CH // 128, 128), jnp.int32),
          pltpu.VMEM((CH // 128, 128), jnp.int32),
          pltpu.VMEM((CH, D), jnp.float32),
          pltpu.VMEM((B // NW,), jnp.int32),
          pltpu.VMEM((B // NW,), jnp.int32),
          pltpu.VMEM((B // NW, D), jnp.float32),
          pltpu.SemaphoreType.DMA,
      ],
      compiler_params=pltpu.CompilerParams(use_tc_tiling_on_sc=False),
  )(huv2d, nodes1d, xv, xu)


def _tc_body(euv, hist, uv, bw1a, br2e, bw1b, bw2, a1b, ba1a, ba2, ba3,
             e5, e16, e16d, t16, seg, sege, b1t, b2t, ab1t, ab2t, ab3t,
             out_ref):
  f32 = jnp.float32
  dot = functools.partial(jnp.dot, preferred_element_type=f32)

  ohv = dot(hist[...], e5[...])
  lane = lax.broadcasted_iota(jnp.int32, (RB, 5 * G), 1) % 5
  oh = jnp.where(ohv == lane.astype(f32), 1.0, 0.0)

  brw = dot(br2e[...], bw1b[...])

  x = dot(euv[...], bw1a[...]) + dot(oh, brw) + b1t[...]
  x = jnp.maximum(x, 0.0)
  o = jnp.maximum(dot(x, bw2[...]) + b2t[...], 0.0)

  uvc = dot(uv[...], a1b[...])
  uvrows = dot(dot(sege[...], uvc), e16d[...])
  a1 = jnp.maximum(dot(o, ba1a[...]) + uvrows + ab1t[...], 0.0)
  a2 = jnp.maximum(dot(a1, ba2[...]) + ab2t[...], 0.0)
  lg = dot(a2, ba3[...]) + ab3t[...]
  ex = jnp.exp(lg)

  owg = dot(o * dot(ex, e16[...]), t16[...])
  num = dot(seg[...], owg)
  den = jnp.sum(dot(seg[...], ex), axis=1, keepdims=True)
  out_ref[...] = num / den


def _tc_call(euv_p, hist_p, uv_rep, consts):
  cspecs = [pl.BlockSpec(c.shape, lambda i, nd=c.ndim: (0,) * nd)
            for c in consts]
  return pl.pallas_call(
      _tc_body,
      grid=(GRID,),
      in_specs=[
          pl.BlockSpec((RB, G * D), lambda i: (i, 0)),
          pl.BlockSpec((RB, G), lambda i: (i, 0)),
          pl.BlockSpec((BB, D), lambda i: (i, 0)),
          *cspecs,
      ],
      out_specs=pl.BlockSpec((BB, D), lambda i: (i, 0)),
      out_shape=jax.ShapeDtypeStruct((B, D), jnp.float32),
      compiler_params=pltpu.CompilerParams(
          dimension_semantics=("arbitrary",)),
  )(euv_p, hist_p, uv_rep, *consts)


def _build_consts(W_r1, b_r1, W_r2, b_r2, r2e,
                  att_w1, att_b1, att_w2, att_b2, att_w3, att_b3):
  i8 = jnp.eye(G, dtype=jnp.float32)
  i16 = jnp.eye(D, dtype=jnp.float32)
  kron = jnp.kron
  return [
      kron(i8, W_r1[:D]),
      kron(i8, r2e),
      kron(i8, W_r1[D:]),
      kron(i8, W_r2),
      att_w1[D:],
      kron(i8, att_w1[:D]),
      kron(i8, att_w2),
      kron(i8, att_w3),
      kron(i8, jnp.ones((1, 5), jnp.float32)),
      kron(i8, jnp.ones((1, D), jnp.float32)),
      kron(jnp.ones((1, G), jnp.float32), i16),
      kron(jnp.ones((G, 1), jnp.float32), i16),
      kron(jnp.eye(BB, dtype=jnp.float32),
           jnp.ones((1, L // G), jnp.float32)),
      kron(jnp.eye(BB, dtype=jnp.float32),
           jnp.ones((L // G, 1), jnp.float32)),
      jnp.tile(b_r1, G)[None],
      jnp.tile(b_r2, G)[None],
      jnp.tile(att_b1, G)[None],
      jnp.tile(att_b2, G)[None],
      jnp.tile(att_b3, G)[None],
  ]


def kernel(nodes, history_uv, history_r, v2e, r2e, u2e, W_r1, b_r1, W_r2,
           b_r2, att_w1, att_b1, att_w2, att_b2, att_w3, att_b3):
  huv2d = history_uv.reshape(P // 128, 128).astype(jnp.int32)
  xv8, xu8 = _repack(v2e.T, u2e.T)
  xv = xv8.reshape(SLOTS, D)
  xu = xu8.reshape(SLOTS, D)
  euv_flat, uv_rep = _sc_gather(huv2d, nodes.astype(jnp.int32), xv, xu)
  euv_p = euv_flat.reshape(ROWS, G * D)
  hist_p = history_r.reshape(ROWS, G).astype(jnp.float32)
  consts = _build_consts(W_r1, b_r1, W_r2, b_r2, r2e,
                         att_w1, att_b1, att_w2, att_b2, att_w3, att_b3)
  return _tc_call(euv_p, hist_p, uv_rep, consts)
